# baseline (speedup 1.0000x reference)
import functools

import jax
import jax.numpy as jnp
from jax import lax
from jax.experimental import pallas as pl
from jax.experimental.pallas import tpu as pltpu
from jax.experimental.pallas import tpu_sc as plsc

_NUM_HEADS = 8
_MAX_SEQ_LEN = 4096
_HEAD_DIM = 128
_TOTAL_SEQ_LEN = 32768
_BATCH = 16

_BLK_T = 512
_BLK_SHIFT = _BLK_T.bit_length() - 1
_NUM_BLK = _TOTAL_SEQ_LEN // _BLK_T

_NC = 2
_NS = 16
_NW = _NC * _NS
_TPT = _TOTAL_SEQ_LEN // _NW


def _splat(v, b):
    return jnp.take_along_axis(v, jnp.full((16,), b, jnp.int32), axis=0)


def _sc_body(lo_hbm, hi_hbm, mask_hbm, bidx_hbm, bval_hbm,
             lo_v, hi_v, mask_v, bidx_v, bval_v):
    cid = lax.axis_index("c")
    sid = lax.axis_index("s")
    wid = cid * _NS + sid

    pltpu.sync_copy(lo_hbm, lo_v)
    pltpu.sync_copy(hi_hbm, hi_v)
    begin = lo_v[...]
    end = hi_v[...]
    stop = jnp.minimum(end, begin + _MAX_SEQ_LEN)

    beg_s = [_splat(begin, b) for b in range(_BATCH)]
    stop_s = [_splat(stop, b) for b in range(_BATCH)]
    iota = lax.iota(jnp.int32, 16)

    one = jnp.full((16,), 1, jnp.int32)
    zero = jnp.full((16,), 0, jnp.int32)
    base = wid * _TPT
    for k in range(_TPT // 16):
        tv = base + k * 16 + iota
        acc = zero
        for b in range(_BATCH):
            inb = jnp.where(tv >= beg_s[b], one, zero) * jnp.where(
                tv < stop_s[b], one, zero)
            acc = jnp.maximum(acc, inb)
        mask_v[pl.ds(k * 16, 16)] = acc.astype(jnp.float32)
    pltpu.sync_copy(mask_v, mask_hbm.at[pl.ds(base, _TPT)])

    @pl.when(wid == 0)
    def _():
        for g in range(_NUM_BLK // 16):
            bs = (g * 16 + iota) * _BLK_T
            be = bs + _BLK_T
            live = zero
            lv = jnp.full((16,), -1, jnp.int32)
            for b in range(_BATCH):
                hit = jnp.where(beg_s[b] < be, one, zero) * jnp.where(
                    stop_s[b] > bs, one, zero)
                live = jnp.maximum(live, hit)
                ok = jnp.where(beg_s[b] < be, one, zero) * jnp.where(
                    stop_s[b] > beg_s[b], one, zero)
                cand = jnp.minimum(stop_s[b], be) - 1
                lv = jnp.maximum(lv, ok * (cand + 1) - 1)
            bidx_v[pl.ds(g * 16, 16)] = lax.shift_right_logical(
                jnp.maximum(lv, 0), _BLK_SHIFT)
            bval_v[pl.ds(g * 16, 16)] = live
        pltpu.sync_copy(bidx_v, bidx_hbm)
        pltpu.sync_copy(bval_v, bval_hbm)


_sc_segment = functools.partial(
    pl.kernel,
    out_type=(
        jax.ShapeDtypeStruct((_TOTAL_SEQ_LEN,), jnp.float32),
        jax.ShapeDtypeStruct((_NUM_BLK,), jnp.int32),
        jax.ShapeDtypeStruct((_NUM_BLK,), jnp.int32),
    ),
    mesh=plsc.VectorSubcoreMesh(core_axis_name="c", subcore_axis_name="s",
                                num_cores=_NC, num_subcores=_NS),
    scratch_types=[
        pltpu.VMEM((_BATCH,), jnp.int32),
        pltpu.VMEM((_BATCH,), jnp.int32),
        pltpu.VMEM((_TPT,), jnp.float32),
        pltpu.VMEM((_NUM_BLK,), jnp.int32),
        pltpu.VMEM((_NUM_BLK,), jnp.int32),
    ],
)(_sc_body)


def _tc_body(bidx_ref, bval_ref, mask_ref, o_ref, do_ref, out_ref):
    i = pl.program_id(0)

    @pl.when(bval_ref[i] != 0)
    def _():
        red = jnp.sum(o_ref[...] * do_ref[...], axis=-1)
        out_ref[...] = red * mask_ref[...]

    @pl.when(bval_ref[i] == 0)
    def _():
        out_ref[...] = jnp.zeros_like(out_ref)


def kernel(o, do, o_offset):
    mask, bidx, bval = _sc_segment(o_offset[:_BATCH], o_offset[1:_BATCH + 1])
    mask2d = mask.reshape(1, _TOTAL_SEQ_LEN)

    grid_spec = pltpu.PrefetchScalarGridSpec(
        num_scalar_prefetch=2,
        grid=(_NUM_BLK,),
        in_specs=[
            pl.BlockSpec((1, _BLK_T), lambda i, bidx, bval: (0, i)),
            pl.BlockSpec((_NUM_HEADS, _BLK_T, _HEAD_DIM),
                         lambda i, bidx, bval: (0, bidx[i], 0)),
            pl.BlockSpec((_NUM_HEADS, _BLK_T, _HEAD_DIM),
                         lambda i, bidx, bval: (0, bidx[i], 0)),
        ],
        out_specs=pl.BlockSpec((_NUM_HEADS, _BLK_T), lambda i, bidx, bval: (0, i)),
    )
    return pl.pallas_call(
        _tc_body,
        grid_spec=grid_spec,
        out_shape=jax.ShapeDtypeStruct((_NUM_HEADS, _TOTAL_SEQ_LEN), jnp.float32),
    )(bidx, bval, mask2d, o, do)


# speedup vs baseline: 1.1629x; 1.1629x over previous
import functools

import jax
import jax.numpy as jnp
from jax import lax
from jax.experimental import pallas as pl
from jax.experimental.pallas import tpu as pltpu
from jax.experimental.pallas import tpu_sc as plsc

_NUM_HEADS = 8
_MAX_SEQ_LEN = 4096
_HEAD_DIM = 128
_TOTAL_SEQ_LEN = 32768
_BATCH = 16

_BLK_T = 512
_BLK_SHIFT = _BLK_T.bit_length() - 1
_NUM_BLK = _TOTAL_SEQ_LEN // _BLK_T

_NC = 2
_NS = 16
_NW = _NC * _NS
_TPT = _TOTAL_SEQ_LEN // _NW


def _splat(v, b):
    return jnp.take_along_axis(v, jnp.full((16,), b, jnp.int32), axis=0)


def _sc_body(lo_hbm, hi_hbm, mask_hbm, bidx_hbm, bval_hbm,
             lo_v, hi_v, mask_v, bidx_v, bval_v):
    cid = lax.axis_index("c")
    sid = lax.axis_index("s")
    wid = cid * _NS + sid

    pltpu.sync_copy(lo_hbm, lo_v)
    pltpu.sync_copy(hi_hbm, hi_v)
    begin = lo_v[...]
    end = hi_v[...]
    stop = jnp.minimum(end, begin + _MAX_SEQ_LEN)

    beg_s = [_splat(begin, b) for b in range(_BATCH)]
    stop_s = [_splat(stop, b) for b in range(_BATCH)]
    iota = lax.iota(jnp.int32, 16)

    one = jnp.full((16,), 1, jnp.int32)
    zero = jnp.full((16,), 0, jnp.int32)
    base = wid * _TPT
    for k in range(_TPT // 16):
        tv = base + k * 16 + iota
        acc = zero
        for b in range(_BATCH):
            inb = jnp.where(tv >= beg_s[b], one, zero) * jnp.where(
                tv < stop_s[b], one, zero)
            acc = jnp.maximum(acc, inb)
        mask_v[pl.ds(k * 16, 16)] = acc.astype(jnp.float32)
    pltpu.sync_copy(mask_v, mask_hbm.at[pl.ds(base, _TPT)])

    @pl.when(wid == 0)
    def _():
        for g in range(_NUM_BLK // 16):
            bs = (g * 16 + iota) * _BLK_T
            be = bs + _BLK_T
            live = zero
            lv = jnp.full((16,), -1, jnp.int32)
            for b in range(_BATCH):
                hit = jnp.where(beg_s[b] < be, one, zero) * jnp.where(
                    stop_s[b] > bs, one, zero)
                live = jnp.maximum(live, hit)
                ok = jnp.where(beg_s[b] < be, one, zero) * jnp.where(
                    stop_s[b] > beg_s[b], one, zero)
                cand = jnp.minimum(stop_s[b], be) - 1
                lv = jnp.maximum(lv, ok * (cand + 1) - 1)
            bidx_v[pl.ds(g * 16, 16)] = lax.shift_right_logical(
                jnp.maximum(lv, 0), _BLK_SHIFT)
            bval_v[pl.ds(g * 16, 16)] = live
        pltpu.sync_copy(bidx_v, bidx_hbm)
        pltpu.sync_copy(bval_v, bval_hbm)


_sc_segment = functools.partial(
    pl.kernel,
    out_type=(
        jax.ShapeDtypeStruct((_TOTAL_SEQ_LEN,), jnp.float32),
        jax.ShapeDtypeStruct((_NUM_BLK,), jnp.int32),
        jax.ShapeDtypeStruct((_NUM_BLK,), jnp.int32),
    ),
    mesh=plsc.VectorSubcoreMesh(core_axis_name="c", subcore_axis_name="s",
                                num_cores=_NC, num_subcores=_NS),
    scratch_types=[
        pltpu.VMEM((_BATCH,), jnp.int32),
        pltpu.VMEM((_BATCH,), jnp.int32),
        pltpu.VMEM((_TPT,), jnp.float32),
        pltpu.VMEM((_NUM_BLK,), jnp.int32),
        pltpu.VMEM((_NUM_BLK,), jnp.int32),
    ],
)(_sc_body)


def _tc_body(bidx_ref, bval_ref, offs_ref, o_ref, do_ref, out_ref):
    i = pl.program_id(0)

    @pl.when(bval_ref[i] != 0)
    def _():
        red = jnp.sum(o_ref[...] * do_ref[...], axis=-1)
        t = i * _BLK_T + jax.lax.broadcasted_iota(
            jnp.int32, (_NUM_HEADS, _BLK_T), 1)
        valid = jnp.zeros((_NUM_HEADS, _BLK_T), dtype=jnp.bool_)
        for b in range(_BATCH):
            begin = offs_ref[b]
            stop = jnp.minimum(offs_ref[b + 1], begin + _MAX_SEQ_LEN)
            valid = valid | ((t >= begin) & (t < stop))
        out_ref[...] = jnp.where(valid, red, 0.0)

    @pl.when(bval_ref[i] == 0)
    def _():
        out_ref[...] = jnp.zeros_like(out_ref)


def kernel(o, do, o_offset):
    mask, bidx, bval = _sc_segment(o_offset[:_BATCH], o_offset[1:_BATCH + 1])
    del mask

    grid_spec = pltpu.PrefetchScalarGridSpec(
        num_scalar_prefetch=3,
        grid=(_NUM_BLK,),
        in_specs=[
            pl.BlockSpec((_NUM_HEADS, _BLK_T, _HEAD_DIM),
                         lambda i, bidx, bval, offs: (0, bidx[i], 0)),
            pl.BlockSpec((_NUM_HEADS, _BLK_T, _HEAD_DIM),
                         lambda i, bidx, bval, offs: (0, bidx[i], 0)),
        ],
        out_specs=pl.BlockSpec((_NUM_HEADS, _BLK_T),
                               lambda i, bidx, bval, offs: (0, i)),
    )
    return pl.pallas_call(
        _tc_body,
        grid_spec=grid_spec,
        out_shape=jax.ShapeDtypeStruct((_NUM_HEADS, _TOTAL_SEQ_LEN), jnp.float32),
    )(bidx, bval, o_offset, o, do)


# speedup vs baseline: 1.4274x; 1.2274x over previous
import functools

import jax
import jax.numpy as jnp
from jax import lax
from jax.experimental import pallas as pl
from jax.experimental.pallas import tpu as pltpu
from jax.experimental.pallas import tpu_sc as plsc

_NUM_HEADS = 8
_MAX_SEQ_LEN = 4096
_HEAD_DIM = 128
_TOTAL_SEQ_LEN = 32768
_BATCH = 16

_BLK_T = 512
_BLK_SHIFT = _BLK_T.bit_length() - 1
_NUM_BLK = _TOTAL_SEQ_LEN // _BLK_T

_NC = 2
_NS = 16
_NW = _NC * _NS
_TPT = _TOTAL_SEQ_LEN // _NW


def _splat(v, b):
    return jnp.take_along_axis(v, jnp.full((16,), b, jnp.int32), axis=0)


def _sc_body(lo_hbm, hi_hbm, mask_hbm, bidx_hbm, bval_hbm,
             lo_v, hi_v, mask_v, bidx_v, bval_v):
    cid = lax.axis_index("c")
    sid = lax.axis_index("s")
    wid = cid * _NS + sid

    pltpu.sync_copy(lo_hbm, lo_v)
    pltpu.sync_copy(hi_hbm, hi_v)
    begin = lo_v[...]
    end = hi_v[...]
    stop = jnp.minimum(end, begin + _MAX_SEQ_LEN)

    beg_s = [_splat(begin, b) for b in range(_BATCH)]
    stop_s = [_splat(stop, b) for b in range(_BATCH)]
    iota = lax.iota(jnp.int32, 16)

    one = jnp.full((16,), 1, jnp.int32)
    zero = jnp.full((16,), 0, jnp.int32)
    base = wid * _TPT
    for k in range(_TPT // 16):
        tv = base + k * 16 + iota
        acc = zero
        for b in range(_BATCH):
            inb = jnp.where(tv >= beg_s[b], one, zero) * jnp.where(
                tv < stop_s[b], one, zero)
            acc = jnp.maximum(acc, inb)
        mask_v[pl.ds(k * 16, 16)] = acc.astype(jnp.float32)
    pltpu.sync_copy(mask_v, mask_hbm.at[pl.ds(base, _TPT)])

    @pl.when(wid == 0)
    def _():
        for g in range(_NUM_BLK // 16):
            bs = (g * 16 + iota) * _BLK_T
            be = bs + _BLK_T
            live = zero
            lv = jnp.full((16,), -1, jnp.int32)
            for b in range(_BATCH):
                hit = jnp.where(beg_s[b] < be, one, zero) * jnp.where(
                    stop_s[b] > bs, one, zero)
                live = jnp.maximum(live, hit)
                ok = jnp.where(beg_s[b] < be, one, zero) * jnp.where(
                    stop_s[b] > beg_s[b], one, zero)
                cand = jnp.minimum(stop_s[b], be) - 1
                lv = jnp.maximum(lv, ok * (cand + 1) - 1)
            bidx_v[pl.ds(g * 16, 16)] = lax.shift_right_logical(
                jnp.maximum(lv, 0), _BLK_SHIFT)
            bval_v[pl.ds(g * 16, 16)] = live
        pltpu.sync_copy(bidx_v, bidx_hbm)
        pltpu.sync_copy(bval_v, bval_hbm)


_sc_segment = functools.partial(
    pl.kernel,
    out_type=(
        jax.ShapeDtypeStruct((_TOTAL_SEQ_LEN,), jnp.float32),
        jax.ShapeDtypeStruct((_NUM_BLK,), jnp.int32),
        jax.ShapeDtypeStruct((_NUM_BLK,), jnp.int32),
    ),
    mesh=plsc.VectorSubcoreMesh(core_axis_name="c", subcore_axis_name="s",
                                num_cores=_NC, num_subcores=_NS),
    scratch_types=[
        pltpu.VMEM((_BATCH,), jnp.int32),
        pltpu.VMEM((_BATCH,), jnp.int32),
        pltpu.VMEM((_TPT,), jnp.float32),
        pltpu.VMEM((_NUM_BLK,), jnp.int32),
        pltpu.VMEM((_NUM_BLK,), jnp.int32),
    ],
)(_sc_body)


def _tc_body(bidx_ref, bval_ref, offs_ref, o_ref, do_ref, out_ref):
    i = pl.program_id(0)

    @pl.when(bval_ref[i] != 0)
    def _():
        red = jnp.sum(o_ref[...] * do_ref[...], axis=-1)
        t = i * _BLK_T + jax.lax.broadcasted_iota(
            jnp.int32, (_NUM_HEADS, _BLK_T), 1)
        valid = jnp.zeros((_NUM_HEADS, _BLK_T), dtype=jnp.bool_)
        for b in range(_BATCH):
            begin = offs_ref[b]
            stop = jnp.minimum(offs_ref[b + 1], begin + _MAX_SEQ_LEN)
            valid = valid | ((t >= begin) & (t < stop))
        out_ref[...] = jnp.where(valid, red, 0.0)

    @pl.when(bval_ref[i] == 0)
    def _():
        out_ref[...] = jnp.zeros_like(out_ref)


def kernel(o, do, o_offset):
    begin = o_offset[:_BATCH]
    end = o_offset[1:_BATCH + 1]
    stop = jnp.minimum(end, begin + _MAX_SEQ_LEN)
    bs = jnp.arange(_NUM_BLK, dtype=jnp.int32) * _BLK_T
    be = bs + _BLK_T
    live = jnp.any((begin[None, :] < be[:, None]) & (stop[None, :] > bs[:, None]), axis=1)
    ok = (begin[None, :] < be[:, None]) & (stop[None, :] > begin[None, :])
    cand = jnp.minimum(stop[None, :], be[:, None]) - 1
    lv = jnp.max(jnp.where(ok, cand, -1), axis=1)
    bidx = (jnp.maximum(lv, 0) >> _BLK_SHIFT).astype(jnp.int32)
    bval = live.astype(jnp.int32)

    grid_spec = pltpu.PrefetchScalarGridSpec(
        num_scalar_prefetch=3,
        grid=(_NUM_BLK,),
        in_specs=[
            pl.BlockSpec((_NUM_HEADS, _BLK_T, _HEAD_DIM),
                         lambda i, bidx, bval, offs: (0, bidx[i], 0)),
            pl.BlockSpec((_NUM_HEADS, _BLK_T, _HEAD_DIM),
                         lambda i, bidx, bval, offs: (0, bidx[i], 0)),
        ],
        out_specs=pl.BlockSpec((_NUM_HEADS, _BLK_T),
                               lambda i, bidx, bval, offs: (0, i)),
    )
    return pl.pallas_call(
        _tc_body,
        grid_spec=grid_spec,
        out_shape=jax.ShapeDtypeStruct((_NUM_HEADS, _TOTAL_SEQ_LEN), jnp.float32),
    )(bidx, bval, o_offset, o, do)


# speedup vs baseline: 1.7022x; 1.1925x over previous
import jax
import jax.numpy as jnp
from jax.experimental import pallas as pl
from jax.experimental.pallas import tpu as pltpu

_NUM_HEADS = 8
_MAX_SEQ_LEN = 4096
_HEAD_DIM = 128
_TOTAL_SEQ_LEN = 32768
_BATCH = 16

_BLK_T = 1024
_NUM_BLK = _TOTAL_SEQ_LEN // _BLK_T


def _tc_body(offs_ref, o_ref, do_ref, out_ref):
    i = pl.program_id(0)
    prod = (o_ref[...] * do_ref[...]).reshape(_NUM_HEADS * _BLK_T, _HEAD_DIM)
    ones = jnp.ones((_HEAD_DIM, 128), dtype=jnp.float32)
    red = jax.lax.dot_general(
        prod, ones, (((1,), (0,)), ((), ())),
        preferred_element_type=jnp.float32)[:, :1]
    red = red.reshape(_NUM_HEADS, _BLK_T)

    t = i * _BLK_T + jax.lax.broadcasted_iota(jnp.int32, (_NUM_HEADS, _BLK_T), 1)
    valid = jnp.zeros((_NUM_HEADS, _BLK_T), dtype=jnp.bool_)
    for b in range(_BATCH):
        begin = offs_ref[b]
        stop = jnp.minimum(offs_ref[b + 1], begin + _MAX_SEQ_LEN)
        valid = valid | ((t >= begin) & (t < stop))
    out_ref[...] = jnp.where(valid, red, 0.0)


def kernel(o, do, o_offset):
    grid_spec = pltpu.PrefetchScalarGridSpec(
        num_scalar_prefetch=1,
        grid=(_NUM_BLK,),
        in_specs=[
            pl.BlockSpec((_NUM_HEADS, _BLK_T, _HEAD_DIM), lambda i, offs: (0, i, 0)),
            pl.BlockSpec((_NUM_HEADS, _BLK_T, _HEAD_DIM), lambda i, offs: (0, i, 0)),
        ],
        out_specs=pl.BlockSpec((_NUM_HEADS, _BLK_T), lambda i, offs: (0, i)),
    )
    return pl.pallas_call(
        _tc_body,
        grid_spec=grid_spec,
        out_shape=jax.ShapeDtypeStruct((_NUM_HEADS, _TOTAL_SEQ_LEN), jnp.float32),
    )(o_offset, o, do)


# speedup vs baseline: 1.7081x; 1.0035x over previous
import jax
import jax.numpy as jnp
from jax.experimental import pallas as pl
from jax.experimental.pallas import tpu as pltpu

_NUM_HEADS = 8
_MAX_SEQ_LEN = 4096
_HEAD_DIM = 128
_TOTAL_SEQ_LEN = 32768
_BATCH = 16

_BLK_T = 2048
_NUM_BLK = _TOTAL_SEQ_LEN // _BLK_T


def _tc_body(offs_ref, o_ref, do_ref, out_ref):
    i = pl.program_id(0)
    prod = (o_ref[...] * do_ref[...]).reshape(_NUM_HEADS * _BLK_T, _HEAD_DIM)
    ones = jnp.ones((_HEAD_DIM, 128), dtype=jnp.float32)
    red = jax.lax.dot_general(
        prod, ones, (((1,), (0,)), ((), ())),
        preferred_element_type=jnp.float32)[:, :1]
    red = red.reshape(_NUM_HEADS, _BLK_T)

    t = i * _BLK_T + jax.lax.broadcasted_iota(jnp.int32, (_NUM_HEADS, _BLK_T), 1)
    valid = jnp.zeros((_NUM_HEADS, _BLK_T), dtype=jnp.bool_)
    for b in range(_BATCH):
        begin = offs_ref[b]
        stop = jnp.minimum(offs_ref[b + 1], begin + _MAX_SEQ_LEN)
        valid = valid | ((t >= begin) & (t < stop))
    out_ref[...] = jnp.where(valid, red, 0.0)


def kernel(o, do, o_offset):
    grid_spec = pltpu.PrefetchScalarGridSpec(
        num_scalar_prefetch=1,
        grid=(_NUM_BLK,),
        in_specs=[
            pl.BlockSpec((_NUM_HEADS, _BLK_T, _HEAD_DIM), lambda i, offs: (0, i, 0)),
            pl.BlockSpec((_NUM_HEADS, _BLK_T, _HEAD_DIM), lambda i, offs: (0, i, 0)),
        ],
        out_specs=pl.BlockSpec((_NUM_HEADS, _BLK_T), lambda i, offs: (0, i)),
    )
    return pl.pallas_call(
        _tc_body,
        grid_spec=grid_spec,
        out_shape=jax.ShapeDtypeStruct((_NUM_HEADS, _TOTAL_SEQ_LEN), jnp.float32),
    )(o_offset, o, do)
